# Initial kernel scaffold; baseline (speedup 1.0000x reference)
import jax
import jax.numpy as jnp
from jax.experimental import pallas as pl


def kernel(x, Wg, W1, b1, W2, b2, Ws1, bs1, Ws2, bs2, gamma, beta):
    raise NotImplementedError("write your pallas kernel here")


# speedup vs baseline: 2.0827x; 2.0827x over previous
import jax
import jax.numpy as jnp
from jax.experimental import pallas as pl
from jax.experimental.pallas import tpu as pltpu

DIM = 1024
E = 8
K = 2
FFN = 2048
N = 2048
EPS = 1e-5
TB = 512
T = N // TB

_INV_SQRT2 = 0.7071067811865476


def _gelu_exact(h):
    return 0.5 * h * (1.0 + jax.lax.erf(h * _INV_SQRT2))


def _dense_kernel(x_ref, wg_ref, w1_ref, b1_ref, w2_ref, b2_ref,
                  gamma_ref, beta_ref, out_ref, acc_ref, comb_ref):
    e = pl.program_id(0)
    t = pl.program_id(1)
    x = x_ref[...]

    @pl.when(e == 0)
    def _init():
        logits = jnp.dot(x, wg_ref[...], preferred_element_type=jnp.float32)
        cols = jax.lax.broadcasted_iota(jnp.int32, (TB, E), 1)
        m1 = jnp.max(logits, axis=1, keepdims=True)
        i1 = jnp.min(jnp.where(logits == m1, cols, E), axis=1, keepdims=True)
        neg = jnp.float32(-jnp.inf)
        logits2 = jnp.where(cols == i1, neg, logits)
        m2 = jnp.max(logits2, axis=1, keepdims=True)
        i2 = jnp.min(jnp.where(logits2 == m2, cols, E), axis=1, keepdims=True)
        s = jnp.exp(m2 - m1)
        w0 = 1.0 / (1.0 + s)
        w1 = s / (1.0 + s)
        ccols = jax.lax.broadcasted_iota(jnp.int32, (TB, E + 1), 1)
        comb = (jnp.where(ccols == i1, w0, 0.0)
                + jnp.where(ccols == i2, w1, 0.0)
                + jnp.where(ccols == E, 1.0, 0.0))
        comb_ref[pl.ds(t * TB, TB), :] = comb
        acc_ref[pl.ds(t * TB, TB), :] = x

    h = jnp.dot(x, w1_ref[0], preferred_element_type=jnp.float32) + b1_ref[0]
    h = _gelu_exact(h)
    y = jnp.dot(h, w2_ref[0], preferred_element_type=jnp.float32) + b2_ref[0]
    ccols = jax.lax.broadcasted_iota(jnp.int32, (TB, E + 1), 1)
    w_e = jnp.sum(jnp.where(ccols == e, comb_ref[pl.ds(t * TB, TB), :], 0.0),
                  axis=1, keepdims=True)
    acc = acc_ref[pl.ds(t * TB, TB), :] + w_e * y
    acc_ref[pl.ds(t * TB, TB), :] = acc

    @pl.when(e == E)
    def _final():
        mu = jnp.mean(acc, axis=1, keepdims=True)
        d = acc - mu
        var = jnp.mean(d * d, axis=1, keepdims=True)
        out_ref[...] = d * jax.lax.rsqrt(var + EPS) * gamma_ref[...] + beta_ref[...]


def kernel(x, Wg, W1, b1, W2, b2, Ws1, bs1, Ws2, bs2, gamma, beta):
    W1a = jnp.concatenate([W1, Ws1[None]], axis=0)
    W2a = jnp.concatenate([W2, Ws2[None]], axis=0)
    b1a = jnp.concatenate([b1, bs1[None]], axis=0)[:, None, :]
    b2a = jnp.concatenate([b2, bs2[None]], axis=0)[:, None, :]
    grid = (E + 1, T)
    out = pl.pallas_call(
        _dense_kernel,
        grid=grid,
        in_specs=[
            pl.BlockSpec((TB, DIM), lambda e, t: (t, 0)),
            pl.BlockSpec((DIM, E), lambda e, t: (0, 0)),
            pl.BlockSpec((1, DIM, FFN), lambda e, t: (e, 0, 0)),
            pl.BlockSpec((1, 1, FFN), lambda e, t: (e, 0, 0)),
            pl.BlockSpec((1, FFN, DIM), lambda e, t: (e, 0, 0)),
            pl.BlockSpec((1, 1, DIM), lambda e, t: (e, 0, 0)),
            pl.BlockSpec((DIM,), lambda e, t: (0,)),
            pl.BlockSpec((DIM,), lambda e, t: (0,)),
        ],
        out_specs=pl.BlockSpec((TB, DIM), lambda e, t: (t, 0)),
        out_shape=jax.ShapeDtypeStruct((N, DIM), jnp.float32),
        scratch_shapes=[
            pltpu.VMEM((N, DIM), jnp.float32),
            pltpu.VMEM((N, E + 1), jnp.float32),
        ],
    )(x, Wg, W1a, b1a, W2a, b2a, gamma, beta)
    return out
